# Initial kernel scaffold; baseline (speedup 1.0000x reference)
import jax
import jax.numpy as jnp
from jax.experimental import pallas as pl


def kernel(x, emb, w1, b1, w2, b2):
    raise NotImplementedError("write your pallas kernel here")


# speedup vs baseline: 8.3649x; 8.3649x over previous
import functools

import jax
import jax.numpy as jnp
from jax import lax
from jax.experimental import pallas as pl
from jax.experimental.pallas import tpu as pltpu
from jax.experimental.pallas import tpu_sc as plsc

B, L, V, D, H, C = 16384, 50, 100000, 128, 256, 1000
CP = 1024
LANES = 16
NC, NS = 2, 16
NW = NC * NS
RPW = B // NW
RG = 8
NG = RPW // RG
DG = D // LANES


def _pool_sc(x, emb):
    mesh = plsc.VectorSubcoreMesh(core_axis_name="c", subcore_axis_name="s")

    @functools.partial(
        pl.kernel,
        out_type=jax.ShapeDtypeStruct((B, D), jnp.float32),
        mesh=mesh,
        scratch_types=[
            pltpu.VMEM((RG, L), jnp.int32),
            pltpu.VMEM((RG, L, D), jnp.float32),
            pltpu.VMEM((RG, D), jnp.float32),
            pltpu.SemaphoreType.DMA,
        ],
    )
    def pool(x_hbm, emb_hbm, out_hbm, idx_v, rows_v, out_v, sem):
        wid = lax.axis_index("s") * NC + lax.axis_index("c")
        row0 = wid * RPW

        def group(g, carry):
            r0 = row0 + g * RG
            pltpu.sync_copy(x_hbm.at[pl.ds(r0, RG), :], idx_v)
            copies = [
                pltpu.async_copy(emb_hbm.at[idx_v.at[r]], rows_v.at[r], sem)
                for r in range(RG)
            ]
            for cp in copies:
                cp.wait()
            for r in range(RG):
                def body(j, accs):
                    return tuple(
                        accs[d] + rows_v[r, j, pl.ds(d * LANES, LANES)]
                        for d in range(DG)
                    )
                accs = lax.fori_loop(
                    0, L, body,
                    tuple(jnp.zeros((LANES,), jnp.float32) for _ in range(DG)),
                )
                for d in range(DG):
                    out_v[r, pl.ds(d * LANES, LANES)] = accs[d] * (1.0 / L)
            pltpu.sync_copy(out_v, out_hbm.at[pl.ds(r0, RG), :])
            return carry

        lax.fori_loop(0, NG, group, 0)

    return pool(x, emb)


def _mlp_tc(pooled, w1, b1, w2p, b2p):
    BM = 1024

    def body(p_ref, w1_ref, b1_ref, w2_ref, b2_ref, o_ref):
        h = jnp.dot(p_ref[...], w1_ref[...], preferred_element_type=jnp.float32)
        h = jnp.maximum(h + b1_ref[...], 0.0)
        logits = jnp.dot(h, w2_ref[...], preferred_element_type=jnp.float32)
        logits = logits + b2_ref[...]
        m = jnp.max(logits, axis=1, keepdims=True)
        z = logits - m
        o_ref[...] = z - jnp.log(jnp.sum(jnp.exp(z), axis=1, keepdims=True))

    return pl.pallas_call(
        body,
        grid=(B // BM,),
        in_specs=[
            pl.BlockSpec((BM, D), lambda i: (i, 0)),
            pl.BlockSpec((D, H), lambda i: (0, 0)),
            pl.BlockSpec((1, H), lambda i: (0, 0)),
            pl.BlockSpec((H, CP), lambda i: (0, 0)),
            pl.BlockSpec((1, CP), lambda i: (0, 0)),
        ],
        out_specs=pl.BlockSpec((BM, CP), lambda i: (i, 0)),
        out_shape=jax.ShapeDtypeStruct((B, CP), jnp.float32),
    )(pooled, w1, b1.reshape(1, H), w2p, b2p.reshape(1, CP))


def kernel(x, emb, w1, b1, w2, b2):
    pooled = _pool_sc(x.astype(jnp.int32), emb)
    w2p = jnp.pad(w2, ((0, 0), (0, CP - C)))
    b2p = jnp.concatenate([b2, jnp.full((CP - C,), -1e30, jnp.float32)])
    out = _mlp_tc(pooled, w1, b1, w2p, b2p)
    return out[:, :C]


# speedup vs baseline: 12.5800x; 1.5039x over previous
import functools

import jax
import jax.numpy as jnp
from jax import lax
from jax.experimental import pallas as pl
from jax.experimental.pallas import tpu as pltpu
from jax.experimental.pallas import tpu_sc as plsc

B, L, V, D, H, C = 16384, 50, 100000, 128, 256, 1000
CP = 1024
LANES = 16
NC, NS = 2, 16
NW = NC * NS
RPW = B // NW
RG = 8
NG = RPW // RG
DG = D // LANES


def _pool_sc(x, emb):
    mesh = plsc.VectorSubcoreMesh(core_axis_name="c", subcore_axis_name="s")

    @functools.partial(
        pl.kernel,
        out_type=jax.ShapeDtypeStruct((B, D), jnp.float32),
        mesh=mesh,
        scratch_types=[
            pltpu.VMEM((2, RG, L), jnp.int32),
            pltpu.VMEM((2, RG, L, D), jnp.float32),
            pltpu.VMEM((RG, D), jnp.float32),
            pltpu.SemaphoreType.DMA,
            pltpu.SemaphoreType.DMA,
        ],
    )
    def pool(x_hbm, emb_hbm, out_hbm, idx_v, rows_v, out_v, sem0, sem1):
        sems = (sem0, sem1)
        wid = lax.axis_index("s") * NC + lax.axis_index("c")
        row0 = wid * RPW

        def fire(slot, g):
            r0 = row0 + g * RG
            pltpu.sync_copy(x_hbm.at[pl.ds(r0, RG), :], idx_v.at[slot])
            for r in range(RG):
                pltpu.async_copy(
                    emb_hbm.at[idx_v.at[slot, r]], rows_v.at[slot, r], sems[slot]
                )

        def drain(slot):
            for r in range(RG):
                pltpu.make_async_copy(
                    emb_hbm.at[idx_v.at[slot, r]], rows_v.at[slot, r], sems[slot]
                ).wait()

        def reduce(slot, g):
            for r in range(RG):
                def body(j, accs):
                    return tuple(
                        accs[d] + rows_v[slot, r, j, pl.ds(d * LANES, LANES)]
                        for d in range(DG)
                    )
                accs = lax.fori_loop(
                    0, L, body,
                    tuple(jnp.zeros((LANES,), jnp.float32) for _ in range(DG)),
                )
                for d in range(DG):
                    out_v[r, pl.ds(d * LANES, LANES)] = accs[d] * (1.0 / L)
            pltpu.sync_copy(out_v, out_hbm.at[pl.ds(row0 + g * RG, RG), :])

        fire(0, 0)

        def body(k, carry):
            g = 2 * k
            fire(1, g + 1)
            drain(0)
            reduce(0, g)
            fire(0, g + 2)
            drain(1)
            reduce(1, g + 1)
            return carry

        lax.fori_loop(0, NG // 2 - 1, body, 0)
        fire(1, NG - 1)
        drain(0)
        reduce(0, NG - 2)
        drain(1)
        reduce(1, NG - 1)

    return pool(x, emb)


def _mlp_tc(pooled, w1, b1, w2p, b2p):
    BM = 1024

    def body(p_ref, w1_ref, b1_ref, w2_ref, b2_ref, o_ref):
        h = jnp.dot(p_ref[...], w1_ref[...], preferred_element_type=jnp.float32)
        h = jnp.maximum(h + b1_ref[...], 0.0)
        logits = jnp.dot(h, w2_ref[...], preferred_element_type=jnp.float32)
        logits = logits + b2_ref[...]
        m = jnp.max(logits, axis=1, keepdims=True)
        z = logits - m
        o_ref[...] = z - jnp.log(jnp.sum(jnp.exp(z), axis=1, keepdims=True))

    return pl.pallas_call(
        body,
        grid=(B // BM,),
        in_specs=[
            pl.BlockSpec((BM, D), lambda i: (i, 0)),
            pl.BlockSpec((D, H), lambda i: (0, 0)),
            pl.BlockSpec((1, H), lambda i: (0, 0)),
            pl.BlockSpec((H, CP), lambda i: (0, 0)),
            pl.BlockSpec((1, CP), lambda i: (0, 0)),
        ],
        out_specs=pl.BlockSpec((BM, CP), lambda i: (i, 0)),
        out_shape=jax.ShapeDtypeStruct((B, CP), jnp.float32),
    )(pooled, w1, b1.reshape(1, H), w2p, b2p.reshape(1, CP))


def kernel(x, emb, w1, b1, w2, b2):
    pooled = _pool_sc(x.astype(jnp.int32), emb)
    w2p = jnp.pad(w2, ((0, 0), (0, CP - C)))
    b2p = jnp.concatenate([b2, jnp.full((CP - C,), -1e30, jnp.float32)])
    out = _mlp_tc(pooled, w1, b1, w2p, b2p)
    return out[:, :C]
